# baseline (speedup 1.0000x reference)
import jax
import jax.numpy as jnp
from jax import lax
from jax.experimental import pallas as pl
from jax.experimental.pallas import tpu as pltpu
from jax.experimental.pallas import tpu_sc as plsc

N_NODES = 10000
N_EDGES = 320000
D = 128

NC = 2
NS = 16

NG = 16
CPG = D // NG
EH = N_EDGES // 2
N_PAD = 10240

CB = 2048
NFULL = EH // CB
TAIL = EH - NFULL * CB

MCW = 1280
NEG_INF = float("-inf")


def _sc_body(values_t, idx_hbm, out_t, scratch, acc, stg, dst_buf, sem_v, sem_d):
    cid = lax.axis_index("c")
    sid = lax.axis_index("s")
    g = cid * (NG // NC) + (sid >> 1)
    h = sid & 1
    gg8 = g * CPG
    ebase = h * EH

    lanes = lax.iota(jnp.int32, 16)
    half01 = lanes >> 3
    half10 = 1 - half01
    colpat = lanes & 7

    def init_body(i, carry):
        for c in range(CPG):
            acc[c, pl.ds(i * 16, 16)] = jnp.full((16,), NEG_INF, jnp.float32)
        return carry

    lax.fori_loop(0, N_PAD // 16, init_body, jnp.int32(0))

    pltpu.async_copy(
        values_t.at[pl.ds(gg8, CPG), pl.ds(ebase, CB)], stg.at[0], sem_v
    )
    pltpu.async_copy(idx_hbm.at[pl.ds(ebase, CB)], dst_buf.at[0], sem_d)

    def do_pairs(b, npairs):
        bvec = jnp.full((16,), b, jnp.int32)

        def pair_body(p, carry):
            for u in range(2):
                pp2 = (p * 2 + u) * 2
                ev = pp2 + half01
                ev_sw = pp2 + half10
                de = plsc.load_gather(dst_buf, [bvec, ev])
                de_sw = plsc.load_gather(dst_buf, [bvec, ev_sw])
                vals = plsc.load_gather(stg, [bvec, colpat, ev])
                vals_sw = plsc.load_gather(stg, [bvec, colpat, ev_sw])
                deq = de == de_sw
                v_eff = jnp.where(deq, jnp.maximum(vals, vals_sw), vals)
                cur = plsc.load_gather(acc, [colpat, de])
                plsc.store_scatter(acc, [colpat, de], jnp.maximum(v_eff, cur))
            return carry

        lax.fori_loop(0, npairs // 2, pair_body, jnp.int32(0))

    def chunk_body(k, carry):
        b = k & 1
        pltpu.make_async_copy(
            values_t.at[pl.ds(gg8, CPG), pl.ds(ebase, CB)], stg.at[0], sem_v
        ).wait()
        pltpu.make_async_copy(
            idx_hbm.at[pl.ds(ebase, CB)], dst_buf.at[0], sem_d
        ).wait()

        @pl.when(k + 1 < NFULL)
        def _():
            off = ebase + (k + 1) * CB
            nb = (k + 1) & 1
            pltpu.async_copy(
                values_t.at[pl.ds(gg8, CPG), pl.ds(off, CB)], stg.at[nb], sem_v
            )
            pltpu.async_copy(idx_hbm.at[pl.ds(off, CB)], dst_buf.at[nb], sem_d)

        do_pairs(b, CB // 2)
        return carry

    lax.fori_loop(0, NFULL, chunk_body, jnp.int32(0))

    pltpu.sync_copy(
        values_t.at[pl.ds(gg8, CPG), pl.ds(ebase + NFULL * CB, TAIL)],
        stg.at[0, pl.ds(0, CPG), pl.ds(0, TAIL)],
    )
    pltpu.sync_copy(
        idx_hbm.at[pl.ds(ebase + NFULL * CB, TAIL)],
        dst_buf.at[0, pl.ds(0, TAIL)],
    )
    do_pairs(jnp.int32(0), TAIL // 2)

    pltpu.sync_copy(acc, scratch.at[g, h])
    plsc.subcore_barrier()

    nh0 = h * (N_PAD // 2)
    for j in range(N_PAD // 2 // MCW):
        noff = nh0 + j * MCW
        pltpu.sync_copy(
            scratch.at[g, 1 - h, pl.ds(0, CPG), pl.ds(noff, MCW)],
            stg.at[0, pl.ds(0, CPG), pl.ds(0, MCW)],
        )

        def merge_body(i, carry):
            for c in range(CPG):
                a = acc[c, pl.ds(noff + i * 16, 16)]
                q = stg[0, c, pl.ds(i * 16, 16)]
                m = jnp.maximum(a, q)
                m = jnp.where(m == NEG_INF, jnp.float32(0), m)
                stg[0, c, pl.ds(i * 16, 16)] = m
            return carry

        lax.fori_loop(0, MCW // 16, merge_body, jnp.int32(0))
        pltpu.sync_copy(
            stg.at[0, pl.ds(0, CPG), pl.ds(0, MCW)],
            out_t.at[pl.ds(gg8, CPG), pl.ds(noff, MCW)],
        )


def _make_agg():
    mesh = plsc.VectorSubcoreMesh(core_axis_name="c", subcore_axis_name="s")
    return pl.kernel(
        _sc_body,
        out_type=(
            jax.ShapeDtypeStruct((D, N_PAD), jnp.float32),
            jax.ShapeDtypeStruct((NG, 2, CPG, N_PAD), jnp.float32),
        ),
        mesh=mesh,
        compiler_params=pltpu.CompilerParams(needs_layout_passes=False),
        scratch_types=[
            pltpu.VMEM((CPG, N_PAD), jnp.float32),
            pltpu.VMEM((2, CPG, CB), jnp.float32),
            pltpu.VMEM((2, CB), jnp.int32),
            pltpu.SemaphoreType.DMA,
            pltpu.SemaphoreType.DMA,
        ],
    )


_agg = _make_agg()


def kernel(source_node_representation_with_coefficient, edge_index):
    idx = edge_index[1]
    values_t = source_node_representation_with_coefficient.T
    out_t, _ = _agg(values_t, idx)
    return out_t.T[:N_NODES]
